# baseline (speedup 1.0000x reference)
import functools
import math

import jax
import jax.numpy as jnp
from jax import lax
from jax.experimental import pallas as pl
from jax.experimental.pallas import tpu as pltpu
from jax.experimental.pallas import tpu_sc as plsc

B = 2
C = 256
H = 64
W = 64
A = 9
NC = 80
SCORE_THRESH = 0.05
NMS_THRESH = 0.5
TOPK = 1000
DETS = 300
IMG = 512
BBOX_CLAMP = math.log(1000.0 / 16.0)

HW = H * W
N = 1024
DPAD = 512


def _post_body(topv_ref, lab_in_ref, bx_ref, sco_ref, lab_ref,
               box_ref, m_ref):
    topv = topv_ref[0]
    labels_f = lab_in_ref[0]
    x1 = bx_ref[0, 0:1]
    y1 = bx_ref[0, 1:2]
    x2 = bx_ref[0, 2:3]
    y2 = bx_ref[0, 3:4]

    valid = topv > SCORE_THRESH

    neg_inf = jnp.float32(-jnp.inf)
    mx = jnp.max(jnp.where(valid, jnp.maximum(jnp.maximum(x1, x2),
                                              jnp.maximum(y1, y2)), neg_inf))
    offs = labels_f * (mx + 1.0)
    sx1, sy1, sx2, sy2 = x1 + offs, y1 + offs, x2 + offs, y2 + offs

    c1 = sx1.reshape(N, 1)
    r1 = sx1
    cy1 = sy1.reshape(N, 1)
    ry1 = sy1
    c2 = sx2.reshape(N, 1)
    r2 = sx2
    cy2 = sy2.reshape(N, 1)
    ry2 = sy2
    areas = (sx2 - sx1) * (sy2 - sy1)
    xx1 = jnp.maximum(c1, r1)
    yy1 = jnp.maximum(cy1, ry1)
    xx2 = jnp.minimum(c2, r2)
    yy2 = jnp.minimum(cy2, ry2)
    inter = jnp.maximum(0.0, xx2 - xx1) * jnp.maximum(0.0, yy2 - yy1)
    iou = inter / (areas.reshape(N, 1) + areas - inter + 1e-12)
    ii = lax.broadcasted_iota(jnp.int32, (N, N), 0)
    jj = lax.broadcasted_iota(jnp.int32, (N, N), 1)
    m_ref[...] = ((iou > NMS_THRESH) & (jj > ii)).astype(jnp.float32)

    idx_row = lax.broadcasted_iota(jnp.int32, (1, N), 1)
    supp0 = 1.0 - valid.astype(jnp.float32)

    def body(i, carry):
        supp, keep = carry
        e_i = (idx_row == i).astype(jnp.float32)
        act = 1.0 - jnp.sum(supp * e_i)
        keep = keep + act * e_i
        row = m_ref[pl.ds(i, 1), :]
        supp = jnp.maximum(supp, act * row)
        return supp, keep

    _, keep = lax.fori_loop(0, TOPK, body,
                            (supp0, jnp.zeros((1, N), jnp.float32)))

    lt = (ii <= jj).astype(jnp.float32)
    rank = jnp.dot(keep, lt, preferred_element_type=jnp.float32,
                   precision=lax.Precision.HIGHEST) - 1.0
    slot = lax.broadcasted_iota(jnp.int32, (1, DPAD), 1).astype(jnp.float32)
    onehot = ((rank.reshape(N, 1) == slot) &
              (keep.reshape(N, 1) > 0.5)).astype(jnp.float32)
    hp = lax.Precision.HIGHEST
    ones = jnp.ones((1, N), jnp.float32)
    stk = jnp.concatenate([topv_ref[0], lab_in_ref[0], ones,
                           x1, y1, x2, y2, ones], axis=0)
    res = jnp.dot(stk, onehot, preferred_element_type=jnp.float32,
                  precision=hp)
    filled = res[7:8]
    sco_ref[0] = res[0:1] + filled - 1.0
    lab_ref[0] = res[1:2] + filled - 1.0
    box_ref[0] = res[3:7]


def _post_call(topv, labf, boxes4):
    return pl.pallas_call(
        _post_body,
        grid=(B,),
        in_specs=[
            pl.BlockSpec((1, 1, N), lambda b: (b, 0, 0)),
            pl.BlockSpec((1, 1, N), lambda b: (b, 0, 0)),
            pl.BlockSpec((1, 4, N), lambda b: (b, 0, 0)),
        ],
        out_specs=[
            pl.BlockSpec((1, 1, DPAD), lambda b: (b, 0, 0)),
            pl.BlockSpec((1, 1, DPAD), lambda b: (b, 0, 0)),
            pl.BlockSpec((1, 4, DPAD), lambda b: (b, 0, 0)),
        ],
        out_shape=[
            jax.ShapeDtypeStruct((B, 1, DPAD), jnp.float32),
            jax.ShapeDtypeStruct((B, 1, DPAD), jnp.float32),
            jax.ShapeDtypeStruct((B, 4, DPAD), jnp.float32),
        ],
        scratch_shapes=[pltpu.VMEM((N, N), jnp.float32)],
    )(topv, labf, boxes4)


ROWD = 128


def _sc_gather(table, idx):
    info = plsc.get_sparse_core_info()
    ncores, nsub = info.num_cores, info.num_subcores
    nw = ncores * nsub
    total = idx.shape[0]
    per_w = total // nw
    mesh = plsc.VectorSubcoreMesh(core_axis_name="c", subcore_axis_name="s")

    @functools.partial(
        pl.kernel, mesh=mesh,
        out_type=jax.ShapeDtypeStruct((total, ROWD), jnp.float32),
        scratch_types=[
            pltpu.VMEM((per_w,), jnp.int32),
            pltpu.VMEM((per_w, ROWD), jnp.float32),
            pltpu.SemaphoreType.DMA,
        ],
    )
    def k(table_hbm, idx_hbm, out_hbm, idx_v, rows_v, sem):
        wid = lax.axis_index("s") * ncores + lax.axis_index("c")
        base = wid * per_w
        pltpu.sync_copy(idx_hbm.at[pl.ds(base, per_w)], idx_v)
        pltpu.async_copy(table_hbm.at[idx_v], rows_v, sem).wait()
        pltpu.sync_copy(rows_v, out_hbm.at[pl.ds(base, per_w)])

    return k(table, idx)


def _conv(x, w, b):
    y = lax.conv_general_dilated(x, w, (1, 1), 'SAME',
                                 dimension_numbers=('NCHW', 'OIHW', 'NCHW'))
    return y + b[None, :, None, None]


def kernel(features, anchors, cls_conv_w, cls_conv_b, cls_out_w, cls_out_b,
           reg_conv_w, reg_conv_b, reg_out_w, reg_out_b):
    t = features
    for i in range(4):
        t = jax.nn.relu(_conv(t, cls_conv_w[i], cls_conv_b[i]))
    logits = _conv(t, cls_out_w, cls_out_b)
    logits = logits.reshape(B, A, NC, H, W).transpose(0, 3, 4, 1, 2)
    t = features
    for i in range(4):
        t = jax.nn.relu(_conv(t, reg_conv_w[i], reg_conv_b[i]))
    reg = _conv(t, reg_out_w, reg_out_b)
    reg = reg.reshape(B, A, 4, H, W).transpose(0, 3, 4, 1, 2).reshape(B, -1, 4)

    scores = jax.nn.sigmoid(logits).reshape(B, -1)

    topv, topi = lax.top_k(scores, TOPK)
    topv = jnp.concatenate(
        [topv, jnp.full((B, N - TOPK), -1.0, jnp.float32)], axis=1)
    topi = jnp.concatenate(
        [topi, jnp.zeros((B, N - TOPK), topi.dtype)], axis=1)
    aidx = topi // NC
    labf = (topi % NC).astype(jnp.float32)
    table = jnp.concatenate(
        [reg.reshape(B, HW * A, 4),
         jnp.broadcast_to(anchors[None], (B, HW * A, 4)),
         jnp.zeros((B, HW * A, ROWD - 8), jnp.float32)],
        axis=2).reshape(B * HW * A, ROWD)
    img_off = (jnp.arange(B, dtype=jnp.int32) * (HW * A))[:, None]
    flat_idx = (aidx + img_off).reshape(B * N)
    rows = _sc_gather(table, flat_idx).reshape(B, N, ROWD)
    reg_g = rows[:, :, 0:4]
    anc_g = rows[:, :, 4:8]
    widths = anc_g[:, :, 2] - anc_g[:, :, 0]
    heights = anc_g[:, :, 3] - anc_g[:, :, 1]
    ctr_x = anc_g[:, :, 0] + 0.5 * widths
    ctr_y = anc_g[:, :, 1] + 0.5 * heights
    dxv, dyv = reg_g[:, :, 0], reg_g[:, :, 1]
    dwv = jnp.minimum(reg_g[:, :, 2], BBOX_CLAMP)
    dhv = jnp.minimum(reg_g[:, :, 3], BBOX_CLAMP)
    pcx = dxv * widths + ctr_x
    pcy = dyv * heights + ctr_y
    pw = jnp.exp(dwv) * widths
    ph = jnp.exp(dhv) * heights
    boxes4 = jnp.clip(
        jnp.stack([pcx - 0.5 * pw, pcy - 0.5 * ph,
                   pcx + 0.5 * pw, pcy + 0.5 * ph], axis=1),
        0.0, float(IMG))

    sco, lab, box = _post_call(topv.reshape(B, 1, N), labf.reshape(B, 1, N),
                               boxes4)

    out_scores = sco[:, 0, :DETS]
    out_labels = lab[:, 0, :DETS].astype(jnp.int32)
    out_boxes = jnp.transpose(box[:, :, :DETS], (0, 2, 1))
    return (out_scores, out_labels, out_boxes)
